# Initial kernel scaffold; baseline (speedup 1.0000x reference)
import jax
import jax.numpy as jnp
from jax.experimental import pallas as pl


def kernel(seq1, seq2, embeddings):
    raise NotImplementedError("write your pallas kernel here")


# speedup vs baseline: 8.1722x; 8.1722x over previous
import functools

import jax
import jax.numpy as jnp
from jax import lax
from jax.experimental import pallas as pl
from jax.experimental.pallas import tpu as pltpu
from jax.experimental.pallas import tpu_sc as plsc

_NUM_CORES = 2
_NUM_SUBCORES = 16
_NUM_WORKERS = _NUM_CORES * _NUM_SUBCORES


def _make_sc_body(L1, L2, D, V, rows_per_w, k_inflight):
    row_w = L2 * D
    win_rows = L2 + rows_per_w - 1
    maxp = (V - 1) // 2

    def body(emb_hbm, out_hbm, embv, win, sem):
        w = lax.axis_index("c") * _NUM_SUBCORES + lax.axis_index("s")
        base = w * rows_per_w
        pltpu.sync_copy(emb_hbm, embv)

        win_start = L1 - 1 - (base + rows_per_w - 1)

        def build(u, c):
            d = win_start + u - (L1 - 1)
            t = jnp.clip(d, -maxp, maxp) + maxp
            for q in range(D // 16):
                win[pl.ds(u * D + q * 16, 16)] = embv[pl.ds(t * D + q * 16, 16)]
            return c

        lax.fori_loop(0, win_rows, build, 0)

        def desc(r):
            return pltpu.make_async_copy(
                win.at[pl.ds((rows_per_w - 1 - r) * D, row_w)],
                out_hbm.at[pl.ds((base + r) * row_w, row_w)],
                sem,
            )

        def group(g, carry):
            r0 = g * k_inflight

            def fire(r, c):
                desc(r0 + r).start()
                return c

            def drain(r, c):
                desc(r0 + r).wait()
                return c

            lax.fori_loop(0, k_inflight, fire, 0)
            lax.fori_loop(0, k_inflight, drain, 0)
            return carry

        lax.fori_loop(0, rows_per_w // k_inflight, group, 0)

    return body


@functools.lru_cache(maxsize=None)
def _make_kernel(L1, L2, D, V):
    assert L1 % _NUM_WORKERS == 0 and D % 16 == 0
    rows_per_w = L1 // _NUM_WORKERS
    k_inflight = 8
    while rows_per_w % k_inflight:
        k_inflight //= 2

    body = _make_sc_body(L1, L2, D, V, rows_per_w, k_inflight)
    sc_call = pl.kernel(
        body,
        out_type=jax.ShapeDtypeStruct((L1 * L2 * D,), jnp.float32),
        mesh=plsc.VectorSubcoreMesh(core_axis_name="c", subcore_axis_name="s"),
        scratch_types=[
            pltpu.VMEM((V * D,), jnp.float32),
            pltpu.VMEM(((L2 + rows_per_w - 1) * D,), jnp.float32),
            pltpu.SemaphoreType.DMA,
        ],
    )

    @jax.jit
    def run(embeddings):
        out = sc_call(embeddings.reshape(-1))
        return out.reshape(L1, L2, D)

    return run


def kernel(seq1, seq2, embeddings):
    L1 = seq1.shape[1]
    L2 = seq2.shape[1]
    V, D = embeddings.shape
    return _make_kernel(L1, L2, D, V)(embeddings)
